# baseline (speedup 1.0000x reference)
import functools

import jax
import jax.numpy as jnp
from jax import lax
from jax.experimental import pallas as pl
from jax.experimental.pallas import tpu as pltpu
from jax.experimental.pallas import tpu_sc as plsc

_LANES = 16
_IDX_CHUNK = 128


@functools.lru_cache(maxsize=None)
def _build(B, D, L1, nc, ns):
    nw = nc * ns
    rows_per = B // nw
    n_chunks = rows_per // _IDX_CHUNK
    n_labels = L1 + 1
    groups = rows_per // 4
    mesh = plsc.VectorSubcoreMesh(core_axis_name="c", subcore_axis_name="s")

    @functools.partial(
        pl.kernel,
        mesh=mesh,
        compiler_params=pltpu.CompilerParams(needs_layout_passes=False,
                                             use_tc_tiling_on_sc=False),
        out_type=jax.ShapeDtypeStruct((B, n_labels), jnp.float32),
        scratch_types=[
            pltpu.VMEM((n_chunks, _IDX_CHUNK), jnp.int32),
            pltpu.VMEM((n_chunks, _IDX_CHUNK), jnp.int32),
            pltpu.VMEM((rows_per, D), jnp.float32),
            pltpu.VMEM((rows_per, D), jnp.float32),
            pltpu.VMEM((rows_per, 1), jnp.float32),
            pltpu.VMEM((rows_per, L1), jnp.float32),
            pltpu.VMEM((rows_per, n_labels), jnp.float32),
            pltpu.VMEM((_LANES, _LANES + 1), jnp.float32),
            pltpu.VMEM((rows_per,), jnp.float32),
            pltpu.SemaphoreType.DMA,
        ],
    )
    def koren_sill(uids_hbm, iids_hbm, uemb_hbm, iemb_hbm, ibias_hbm,
                   ubeta_hbm, out_hbm, uidx, iidx, urows, irows, bias, beta,
                   outbuf, accbuf, ybuf, sem):
        wid = lax.axis_index("s") * nc + lax.axis_index("c")
        base_chunk = wid * n_chunks

        pltpu.sync_copy(uids_hbm.at[pl.ds(base_chunk, n_chunks)], uidx)
        pltpu.sync_copy(iids_hbm.at[pl.ds(base_chunk, n_chunks)], iidx)

        copies = []
        for j in range(n_chunks):
            r0 = j * _IDX_CHUNK
            sl = pl.ds(r0, _IDX_CHUNK)
            copies.append(pltpu.async_copy(uemb_hbm.at[uidx.at[j]], urows.at[sl], sem))
            copies.append(pltpu.async_copy(iemb_hbm.at[iidx.at[j]], irows.at[sl], sem))
            copies.append(pltpu.async_copy(ibias_hbm.at[iidx.at[j]], bias.at[sl], sem))
            copies.append(pltpu.async_copy(ubeta_hbm.at[uidx.at[j]], beta.at[sl], sem))
        for c in copies:
            c.wait()

        lane = lax.iota(jnp.int32, _LANES)
        kv = lane & 3
        dv = lane >> 2
        zero16 = jnp.zeros((_LANES,), jnp.int32)
        zf = jnp.zeros((_LANES,), jnp.float32)

        def dot_body(blk, carry):
            for rr in range(_LANES):
                r = blk * _LANES + rr
                acc = urows[r, pl.ds(0, _LANES)] * irows[r, pl.ds(0, _LANES)]
                for c0 in range(_LANES, D, _LANES):
                    acc = acc + urows[r, pl.ds(c0, _LANES)] * irows[r, pl.ds(c0, _LANES)]
                accbuf[rr, pl.ds(0, _LANES)] = acc
            y16 = plsc.load_gather(accbuf, [lane, zero16])
            for c0 in range(1, _LANES):
                y16 = y16 + plsc.load_gather(accbuf, [lane, zero16 + c0])
            ybuf[pl.ds(blk * _LANES, _LANES)] = y16
            return carry

        lax.fori_loop(0, rows_per // _LANES, dot_body, 0)

        def group_body(g, carry):
            rows16 = g * 4 + dv
            bias_v = plsc.load_gather(bias, [rows16, zero16])
            yv = plsc.load_gather(ybuf, [rows16]) + bias_v
            b0 = plsc.load_gather(beta, [rows16, zero16])
            e1 = jnp.exp(plsc.load_gather(beta, [rows16, zero16 + 1]))
            e2 = jnp.exp(plsc.load_gather(beta, [rows16, zero16 + 2]))
            e3 = jnp.exp(plsc.load_gather(beta, [rows16, zero16 + 3]))
            t_cur = (b0 + jnp.where(kv >= 1, e1, zf)
                     + jnp.where(kv >= 2, e2, zf) + jnp.where(kv >= 3, e3, zf))
            t_prev = b0 + jnp.where(kv >= 2, e1, zf) + jnp.where(kv >= 3, e2, zf)
            s_cur = 1.0 / (1.0 + jnp.exp(yv - t_cur))
            s_prev = jnp.where(kv == 0, zf, 1.0 / (1.0 + jnp.exp(yv - t_prev)))
            plsc.store_scatter(outbuf, [rows16, kv], s_cur - s_prev)
            plsc.store_scatter(outbuf, [rows16, zero16 + 4], 1.0 - s_cur,
                               mask=(kv == 3))
            return carry

        lax.fori_loop(0, groups, group_body, 0)

        pltpu.sync_copy(outbuf, out_hbm.at[pl.ds(wid * rows_per, rows_per)])

    return koren_sill


def kernel(user_ids, item_ids, user_emb_w, item_emb_w, item_bias_w, user_beta_w):
    B = user_ids.shape[0]
    D = user_emb_w.shape[1]
    L1 = user_beta_w.shape[1]
    info = plsc.get_sparse_core_info()
    k = _build(B, D, L1, info.num_cores, info.num_subcores)
    uids2 = user_ids.reshape(-1, _IDX_CHUNK)
    iids2 = item_ids.reshape(-1, _IDX_CHUNK)
    return k(uids2, iids2, user_emb_w, item_emb_w, item_bias_w, user_beta_w)
